# baseline (speedup 1.0000x reference)
import functools

import jax
import jax.numpy as jnp
from jax import lax
from jax.experimental import pallas as pl
from jax.experimental.pallas import tpu as pltpu
from jax.experimental.pallas import tpu_sc as plsc

N_NODES = 10000
N_EDGES = 320000
N_GRAPHS = 512
D_FEAT = 128
D_EDGE = 16
N_TILES = 32

NCH = 128
N_FULL = N_NODES // NCH
N_REM = N_NODES - N_FULL * NCH

EK = 512
E_CHUNKS = N_EDGES // EK
K_MAX = -(-E_CHUNKS // N_TILES)
EG = EK // 16
EROWS = D_EDGE * N_GRAPHS // 128
ACC_LEN = D_EDGE * N_GRAPHS


def _sc_body(x_hbm, batch_hbm, col_hbm, eat_hbm, node_out, edge_out,
             accn_sp, batch_v, col_v, et_v, col2_v, et2_v, acc2_v,
             bidx_v, xv, bidx1_v, xv1, bidx16_v, x16_v, zn_v,
             sem0, sem1, semb, semn0, semn1, semns0, semns1):
    cid = lax.axis_index("c")
    sid = lax.axis_index("s")
    wid = cid * 16 + sid

    col_bufs = (col_v, col2_v)
    et_bufs = (et_v, et2_v)
    sems = (sem0, sem1)
    node_bufs = ((bidx_v, xv, semn0), (bidx1_v, xv1, semn1))

    def start_round(k):
        base = (wid + N_TILES * k) * EK
        s = sems[k % 2]
        return (pltpu.async_copy(col_hbm.at[pl.ds(base, EK)],
                                 col_bufs[k % 2], s),
                pltpu.async_copy(eat_hbm.at[:, pl.ds(base, EK)],
                                 et_bufs[k % 2], s))

    def start_node(k):
        bb, xb, s = node_bufs[k % 2]
        base = (wid + N_TILES * k) * NCH
        return (pltpu.async_copy(batch_hbm.at[pl.ds(base, NCH)], bb, s),
                pltpu.async_copy(x_hbm.at[pl.ds(base, NCH), :], xb, s))

    bd = pltpu.async_copy(batch_hbm, batch_v, semb)
    pend0 = start_round(0)
    pend1 = start_round(1)
    nd0 = start_node(0)

    zero16 = jnp.zeros((16,), jnp.float32)
    for r in range(EROWS * D_FEAT // 16):
        acc2_v[pl.ds(r * 16, 16)] = zero16
    for r in range(32):
        for c in range(D_FEAT // 16):
            zn_v[r, pl.ds(c * 16, 16)] = zero16
    pltpu.sync_copy(zn_v, accn_sp.at[pl.ds(sid * 32, 32)])

    bd.wait()
    plsc.subcore_barrier()

    N_FULL_ROUNDS = E_CHUNKS // N_TILES
    pend = [pend0, pend1]
    nd = [nd0, None]
    ns = [None, None, None]

    def compute_round(k):
        cv, ev = col_bufs[k % 2], et_bufs[k % 2]

        def one_group(j):
            idx = cv[pl.ds(j * 16, 16)]
            sv = plsc.load_gather(batch_v, [idx])
            addr = sv
            for f in range(D_EDGE):
                vals = ev[f, pl.ds(j * 16, 16)]
                plsc.addupdate_scatter(acc2_v, [addr], vals)
                if f + 1 < D_EDGE:
                    addr = addr + N_GRAPHS

        def group(j, carry):
            one_group(j * 2)
            one_group(j * 2 + 1)
            return carry

        lax.fori_loop(0, EG // 2, group, 0)

    for k in range(N_FULL_ROUNDS):
        for d in pend[k % 2]:
            d.wait()
        compute_round(k)
        if k + 2 < N_FULL_ROUNDS:
            pend[k % 2] = start_round(k + 2)
        if k == 0:
            nd[1] = start_node(1)
            for d in nd[0]:
                d.wait()
            ns[0] = pltpu.async_copy(xv, accn_sp.at[bidx_v], semns0, add=True)
        elif k == 1:
            for d in nd[1]:
                d.wait()
            ns[1] = pltpu.async_copy(xv1, accn_sp.at[bidx1_v], semns1,
                                     add=True)
            ns[0].wait()

            @pl.when(wid < 14)
            def _():
                for d in start_node(2):
                    d.wait()

        elif k == 2:

            @pl.when(wid < 14)
            def _():
                pltpu.sync_copy(xv, accn_sp.at[bidx_v], add=True)

            @pl.when(wid == 14)
            def _():
                base = N_FULL * NCH
                pltpu.sync_copy(batch_hbm.at[pl.ds(base, N_REM)], bidx16_v)
                pltpu.sync_copy(x_hbm.at[pl.ds(base, N_REM), :], x16_v)
                pltpu.sync_copy(x16_v, accn_sp.at[bidx16_v], add=True)

    ns[1].wait()

    @pl.when(wid + N_TILES * N_FULL_ROUNDS < E_CHUNKS)
    def _():
        base = (wid + N_TILES * N_FULL_ROUNDS) * EK
        kb = N_FULL_ROUNDS % 2
        pltpu.sync_copy(col_hbm.at[pl.ds(base, EK)], col_bufs[kb])
        pltpu.sync_copy(eat_hbm.at[:, pl.ds(base, EK)], et_bufs[kb])
        compute_round(N_FULL_ROUNDS)

    pltpu.sync_copy(acc2_v, edge_out.at[pl.ds(wid * ACC_LEN, ACC_LEN)])

    plsc.subcore_barrier()
    pltpu.sync_copy(accn_sp.at[pl.ds(sid * 32, 32)],
                    node_out.at[cid, pl.ds(sid * 32, 32)])


_sc_aggregate = pl.kernel(
    _sc_body,
    out_type=(
        jax.ShapeDtypeStruct((2, N_GRAPHS, D_FEAT), jnp.float32),
        jax.ShapeDtypeStruct((N_TILES * ACC_LEN,), jnp.float32),
    ),
    mesh=plsc.VectorSubcoreMesh(core_axis_name="c", subcore_axis_name="s"),
    compiler_params=pltpu.CompilerParams(needs_layout_passes=False,
                                         use_tc_tiling_on_sc=True),
    scratch_types=[
        pltpu.VMEM_SHARED((N_GRAPHS, D_FEAT), jnp.float32),
        pltpu.VMEM((N_NODES,), jnp.int32),
        pltpu.VMEM((EK,), jnp.int32),
        pltpu.VMEM((D_EDGE, EK), jnp.float32),
        pltpu.VMEM((EK,), jnp.int32),
        pltpu.VMEM((D_EDGE, EK), jnp.float32),
        pltpu.VMEM((ACC_LEN,), jnp.float32),
        pltpu.VMEM((NCH,), jnp.int32),
        pltpu.VMEM((NCH, D_FEAT), jnp.float32),
        pltpu.VMEM((NCH,), jnp.int32),
        pltpu.VMEM((NCH, D_FEAT), jnp.float32),
        pltpu.VMEM((N_REM,), jnp.int32),
        pltpu.VMEM((N_REM, D_FEAT), jnp.float32),
        pltpu.VMEM((32, D_FEAT), jnp.float32),
        pltpu.SemaphoreType.DMA,
        pltpu.SemaphoreType.DMA,
        pltpu.SemaphoreType.DMA,
        pltpu.SemaphoreType.DMA,
        pltpu.SemaphoreType.DMA,
        pltpu.SemaphoreType.DMA,
        pltpu.SemaphoreType.DMA,
    ],
)


def _mlp_body(np_ref, et_ref, w1_ref, b1_ref, g1_ref, bt1_ref,
              w2_ref, b2_ref, g2_ref, bt2_ref, w3_ref, b3_ref, o_ref):
    f32 = jnp.float32
    hi = jax.lax.Precision.HIGHEST
    node = np_ref[0] + np_ref[1]
    edge_t = et_ref[...]
    w1 = w1_ref[...]
    h = (jnp.dot(node, w1[:D_FEAT], preferred_element_type=f32, precision=hi)
         + lax.dot_general(edge_t, w1[D_FEAT:],
                           (((0,), (0,)), ((), ())),
                           preferred_element_type=f32, precision=hi)
         + b1_ref[...])
    h = jnp.where(h >= 0, h, 0.01 * h)
    mean = jnp.mean(h, axis=0, keepdims=True)
    d = h - mean
    var = jnp.mean(d * d, axis=0, keepdims=True)
    h = g1_ref[...] * d / jnp.sqrt(var + 1e-5) + bt1_ref[...]

    h = jnp.dot(h, w2_ref[...], preferred_element_type=f32, precision=hi) + b2_ref[...]
    h = jnp.where(h >= 0, h, 0.01 * h)
    mean = jnp.mean(h, axis=0, keepdims=True)
    d = h - mean
    var = jnp.mean(d * d, axis=0, keepdims=True)
    h = g2_ref[...] * d / jnp.sqrt(var + 1e-5) + bt2_ref[...]

    o_ref[...] = jnp.dot(h, w3_ref[...], preferred_element_type=f32,
                         precision=hi) + b3_ref[...]


_mlp_head = pl.pallas_call(
    _mlp_body,
    out_shape=jax.ShapeDtypeStruct((N_GRAPHS, D_FEAT), jnp.float32),
)


@functools.partial(jax.jit, static_argnames=())
def kernel(x, edge_index, edge_attr, u, batch, W1, b1, g1, beta1,
           W2, b2, g2, beta2, W3, b3):
    col = edge_index[1].astype(jnp.int32)
    batch32 = batch.astype(jnp.int32)
    ea_t = edge_attr.T
    node_part, edge_part = _sc_aggregate(x, batch32, col, ea_t)
    edge_t = edge_part.reshape(N_TILES, D_EDGE, N_GRAPHS).sum(axis=0)
    return _mlp_head(node_part, edge_t,
                     W1, b1.reshape(1, -1), g1.reshape(1, -1),
                     beta1.reshape(1, -1),
                     W2, b2.reshape(1, -1), g2.reshape(1, -1),
                     beta2.reshape(1, -1),
                     W3, b3.reshape(1, -1))
